# baseline (speedup 1.0000x reference)
import jax
import jax.numpy as jnp
from jax import lax
from jax.experimental import pallas as pl
from jax.experimental.pallas import tpu as pltpu
from jax.experimental.pallas import tpu_sc as plsc
from jax.experimental.compute_on import compute_on

B = 4
S = 4096
D = 1024
NC = 2
NS = 16
NW = NC * NS
SC_S = 1024
TC_S = S - SC_S
CHUNK = SC_S // NW
ROWS = 8
NBLK = CHUNK // ROWS
NVEC = D // 16

TC_SB = 512


def _sc_body(x_hbm, scale_hbm, pos_hbm, tse_hbm, out_hbm,
             xbuf, posbuf, trow, idxv, isem0, isem1, osem0, osem1, gsem):
    wid = lax.axis_index("s") * NC + lax.axis_index("c")
    base = wid * CHUNK

    pltpu.sync_copy(scale_hbm, idxv)
    pltpu.async_copy(tse_hbm.at[idxv], trow, gsem).wait()

    def in_descs(buf, blk):
        s0 = base + blk * ROWS
        return (
            pltpu.make_async_copy(pos_hbm.at[pl.ds(TC_S + s0, ROWS)],
                                  posbuf.at[buf], isem[buf]),
            pltpu.make_async_copy(x_hbm.at[:, pl.ds(TC_S + s0, ROWS)],
                                  xbuf.at[buf], isem[buf]),
        )

    isem = (isem0, isem1)
    osem = (osem0, osem1)

    def out_desc(buf, blk):
        s0 = base + blk * ROWS
        return pltpu.make_async_copy(xbuf.at[buf],
                                     out_hbm.at[:, pl.ds(s0, ROWS)],
                                     osem[buf])

    def fire_in(buf, blk):
        for d in in_descs(buf, blk):
            d.start()

    def drain_in(buf, blk):
        for d in in_descs(buf, blk):
            d.wait()

    def compute(buf):
        def jblock(j, carry):
            dj = pl.ds(pl.multiple_of(j * 16, 16), 16)
            tv = trow[0, dj]
            pvs = [posbuf[buf, s, dj] + tv for s in range(ROWS)]
            for s in range(ROWS):
                for b in range(B):
                    plsc.addupdate(xbuf.at[buf, b, s, dj], pvs[s])
            return carry
        lax.fori_loop(0, NVEC, jblock, 0)

    fire_in(0, 0)

    def step(it, carry):
        for phase in range(2):
            blk = it * 2 + phase
            cur, nxt = phase, 1 - phase

            @pl.when(blk >= 1)
            def _():
                out_desc(nxt, blk - 1).wait()

            @pl.when(blk + 1 < NBLK)
            def _():
                fire_in(nxt, blk + 1)

            drain_in(cur, blk)
            compute(cur)
            out_desc(cur, blk).start()
        return carry

    lax.fori_loop(0, NBLK // 2, step, 0)
    out_desc((NBLK - 1) % 2, NBLK - 1).wait()


def _run_sc(x, scale_arr, pos_embedding, tse):
    mesh = plsc.VectorSubcoreMesh(core_axis_name="c", subcore_axis_name="s")
    kfn = pl.kernel(
        _sc_body,
        out_type=jax.ShapeDtypeStruct((B, SC_S, D), jnp.float32),
        mesh=mesh,
        scratch_types=[
            pltpu.VMEM((2, B, ROWS, D), jnp.float32),
            pltpu.VMEM((2, ROWS, D), jnp.float32),
            pltpu.VMEM((1, D), jnp.float32),
            pltpu.VMEM((1,), jnp.int32),
            pltpu.SemaphoreType.DMA,
            pltpu.SemaphoreType.DMA,
            pltpu.SemaphoreType.DMA,
            pltpu.SemaphoreType.DMA,
            pltpu.SemaphoreType.DMA,
        ],
    )
    return kfn(x, scale_arr, pos_embedding, tse)


TC_NSTEP = TC_S // TC_SB


def _tc_body(scale_ref, x_any, pos_any, tse_any, out_any,
             xb, posb, tseb, isem0, isem1, osem0, osem1, tsem):
    isem = (isem0, isem1)
    osem = (osem0, osem1)

    tse_copy = pltpu.make_async_copy(tse_any, tseb, tsem)
    tse_copy.start()
    tse_copy.wait()
    idx = scale_ref[0]
    trow = tseb[idx, :]

    def in_descs(buf, step):
        s0 = step * TC_SB
        ds = [pltpu.make_async_copy(x_any.at[b, pl.ds(s0, TC_SB)],
                                    xb.at[buf, b], isem[buf])
              for b in range(B)]
        ds.append(pltpu.make_async_copy(pos_any.at[pl.ds(s0, TC_SB)],
                                        posb.at[buf], isem[buf]))
        return ds

    def out_descs(buf, step):
        s0 = step * TC_SB
        return [pltpu.make_async_copy(xb.at[buf, b],
                                      out_any.at[b, pl.ds(s0, TC_SB)],
                                      osem[buf])
                for b in range(B)]

    def fire(descs):
        for d in descs:
            d.start()

    def drain(descs):
        for d in descs:
            d.wait()

    def compute(buf):
        pv = posb[buf] + trow[None, :]
        for b in range(B):
            xb[buf, b] = xb[buf, b] + pv

    fire(in_descs(0, 0))

    def step_fn(it, carry):
        for phase in range(2):
            stp = it * 2 + phase
            cur, nxt = phase, 1 - phase

            @pl.when(stp >= 1)
            def _():
                drain(out_descs(nxt, stp - 1))

            @pl.when(stp + 1 < TC_NSTEP)
            def _():
                fire(in_descs(nxt, stp + 1))

            drain(in_descs(cur, stp))
            compute(cur)
            fire(out_descs(cur, stp))
        return carry

    lax.fori_loop(0, TC_NSTEP // 2, step_fn, 0)
    drain(out_descs((TC_NSTEP - 1) % 2, TC_NSTEP - 1))


def _run_tc(x, scale_arr, pos_embedding, tse):
    return pl.pallas_call(
        _tc_body,
        in_specs=[
            pl.BlockSpec(memory_space=pltpu.SMEM),
            pl.BlockSpec(memory_space=pl.ANY),
            pl.BlockSpec(memory_space=pl.ANY),
            pl.BlockSpec(memory_space=pl.ANY),
        ],
        out_specs=pl.BlockSpec(memory_space=pl.ANY),
        out_shape=jax.ShapeDtypeStruct((B, S, D), jnp.float32),
        scratch_shapes=[
            pltpu.VMEM((2, B, TC_SB, D), jnp.float32),
            pltpu.VMEM((2, TC_SB, D), jnp.float32),
            pltpu.VMEM((10, D), jnp.float32),
            pltpu.SemaphoreType.DMA,
            pltpu.SemaphoreType.DMA,
            pltpu.SemaphoreType.DMA,
            pltpu.SemaphoreType.DMA,
            pltpu.SemaphoreType.DMA,
        ],
    )(scale_arr, x, pos_embedding, tse)


@jax.jit
def _run(x, scale_arr, pos_embedding, tse):
    with compute_on("tpu_sparsecore"):
        sc_out = _run_sc(x, scale_arr, pos_embedding, tse)
    tc_out = _run_tc(x, scale_arr, pos_embedding, tse)
    tc_out, sc_out = lax.optimization_barrier((tc_out, sc_out))
    return lax.dynamic_update_slice(tc_out, sc_out, (0, TC_S, 0))


def kernel(x, time_scale, pos_embedding, time_scale_embedding):
    ts = jnp.asarray(time_scale).astype(jnp.float32)
    scale_idx = jnp.minimum(jnp.log2(ts).astype(jnp.int32), 9)
    scale_arr = scale_idx.reshape((1,)).astype(jnp.int32)
    return _run(x, scale_arr, pos_embedding, time_scale_embedding)


# speedup vs baseline: 1.0356x; 1.0356x over previous
import jax
import jax.numpy as jnp
from jax import lax
from jax.experimental import pallas as pl
from jax.experimental.pallas import tpu as pltpu
from jax.experimental.pallas import tpu_sc as plsc

B = 4
S = 4096
D = 1024
NC = 2
NS = 16
NW = NC * NS
CHUNK = S // NW
ROWS = 8
NBLK = CHUNK // ROWS
NVEC = D // 16


def _body(x_hbm, scale_hbm, pos_hbm, tse_hbm, out_hbm,
          xbuf, posbuf, trow, idxv, isem0, isem1, osem0, osem1, gsem):
    wid = lax.axis_index("s") * NC + lax.axis_index("c")
    base = wid * CHUNK
    isem = (isem0, isem1)
    osem = (osem0, osem1)

    pltpu.sync_copy(scale_hbm, idxv)
    pltpu.async_copy(tse_hbm.at[idxv], trow, gsem).wait()

    def in_descs(buf, blk):
        s0 = base + blk * ROWS
        return (
            pltpu.make_async_copy(pos_hbm.at[pl.ds(s0, ROWS)],
                                  posbuf.at[buf], isem[buf]),
            pltpu.make_async_copy(x_hbm.at[:, pl.ds(s0, ROWS)],
                                  xbuf.at[buf], isem[buf]),
        )

    def out_desc(buf, blk):
        s0 = base + blk * ROWS
        return pltpu.make_async_copy(xbuf.at[buf],
                                     out_hbm.at[:, pl.ds(s0, ROWS)],
                                     osem[buf])

    def fire_in(buf, blk):
        for d in in_descs(buf, blk):
            d.start()

    def drain_in(buf, blk):
        for d in in_descs(buf, blk):
            d.wait()

    def compute(buf):
        def jblock(j, carry):
            dj = pl.ds(pl.multiple_of(j * 16, 16), 16)
            tv = trow[0, dj]
            pvs = [posbuf[buf, s, dj] + tv for s in range(ROWS)]
            for s in range(ROWS):
                for b in range(B):
                    plsc.addupdate(xbuf.at[buf, b, s, dj], pvs[s])
            return carry
        lax.fori_loop(0, NVEC, jblock, 0)

    fire_in(0, 0)

    def step(it, carry):
        for phase in range(2):
            blk = it * 2 + phase
            cur, nxt = phase, 1 - phase

            @pl.when(blk >= 1)
            def _():
                out_desc(nxt, blk - 1).wait()

            @pl.when(blk + 1 < NBLK)
            def _():
                fire_in(nxt, blk + 1)

            drain_in(cur, blk)
            compute(cur)
            out_desc(cur, blk).start()
        return carry

    lax.fori_loop(0, NBLK // 2, step, 0)
    out_desc((NBLK - 1) % 2, NBLK - 1).wait()


@jax.jit
def _run(x, scale_arr, pos_embedding, tse):
    mesh = plsc.VectorSubcoreMesh(core_axis_name="c", subcore_axis_name="s")
    kfn = pl.kernel(
        _body,
        out_type=jax.ShapeDtypeStruct((B, S, D), jnp.float32),
        mesh=mesh,
        scratch_types=[
            pltpu.VMEM((2, B, ROWS, D), jnp.float32),
            pltpu.VMEM((2, ROWS, D), jnp.float32),
            pltpu.VMEM((1, D), jnp.float32),
            pltpu.VMEM((1,), jnp.int32),
            pltpu.SemaphoreType.DMA,
            pltpu.SemaphoreType.DMA,
            pltpu.SemaphoreType.DMA,
            pltpu.SemaphoreType.DMA,
            pltpu.SemaphoreType.DMA,
        ],
    )
    return kfn(x, scale_arr, pos_embedding, tse)


def kernel(x, time_scale, pos_embedding, time_scale_embedding):
    ts = jnp.asarray(time_scale).astype(jnp.float32)
    scale_idx = jnp.minimum(jnp.log2(ts).astype(jnp.int32), 9)
    scale_arr = scale_idx.reshape((1,)).astype(jnp.int32)
    return _run(x, scale_arr, pos_embedding, time_scale_embedding)
